# baseline (speedup 1.0000x reference)
import functools

import jax
import jax.numpy as jnp
from jax import lax
from jax.experimental import pallas as pl
from jax.experimental.pallas import tpu as pltpu
from jax.experimental.pallas import tpu_sc as plsc

N_DOMAIN = 26
VOCAB = 100000
DIM = 64
BATCH = 16384
NC, NS, L = 2, 16, 16
NW = NC * NS
PAIRS = N_DOMAIN * DIM
P_PER_W = PAIRS // NW
HALF_B = BATCH // 2


QVEC = BATCH // 4 // L


def _sc_body(tab_hbm, cond_hbm, out_hbm, row_v, idx_v, outc0, outc1,
             sem_r, sem_o0, sem_o1):
    wid = lax.axis_index("s") * NC + lax.axis_index("c")
    pair0 = wid * P_PER_W
    outc = (outc0, outc1)
    sems = (sem_o0, sem_o1)

    def do_pair(p, _):
        pair = pair0 + p
        i = pair // DIM
        d = pair - i * DIM
        h = d // 8
        w = d - h * 8

        @pl.when((d == 0) | (p == 0))
        def _():
            pltpu.async_copy(cond_hbm.at[i, :], idx_v, sem_r).wait()

        pltpu.async_copy(tab_hbm.at[i, d, :], row_v, sem_r).wait()

        handles = [None, None]
        for q in range(4):
            buf = outc[q % 2]
            if handles[q % 2] is not None:
                handles[q % 2].wait()

            @plsc.parallel_loop(0, 32, unroll=2)
            def brow(r, _q=q, _buf=buf):
                base = _q * (QVEC * L) + r * 128
                for k in range(8):
                    v = idx_v[pl.ds(base + k * L, L)]
                    g = plsc.load_gather(row_v, [v])
                    _buf[r, pl.ds(k * L, L)] = g
            handles[q % 2] = pltpu.async_copy(
                buf, out_hbm.at[i, h, pl.ds(q * 32, 32), w, :], sems[q % 2])
        handles[0].wait()
        handles[1].wait()
        return 0

    lax.fori_loop(0, P_PER_W, do_pair, 0)


@functools.cache
def _gather_call():
    return functools.partial(
        pl.kernel,
        mesh=plsc.VectorSubcoreMesh(core_axis_name="c", subcore_axis_name="s",
                                    num_cores=NC),
        out_type=jax.ShapeDtypeStruct((N_DOMAIN, 8, 128, 8, 128), jnp.float32),
        compiler_params=pltpu.CompilerParams(use_tc_tiling_on_sc=True,
                                             needs_layout_passes=False),
        scratch_types=[
            pltpu.VMEM((VOCAB,), jnp.float32),
            pltpu.VMEM((BATCH,), jnp.int32),
            pltpu.VMEM((32, 128), jnp.float32),
            pltpu.VMEM((32, 128), jnp.float32),
            pltpu.SemaphoreType.DMA,
            pltpu.SemaphoreType.DMA,
            pltpu.SemaphoreType.DMA,
        ],
    )(_sc_body)


def kernel(conditions, tables):
    tab_t = jnp.transpose(tables, (0, 2, 1))
    cond_t = jnp.transpose(conditions.astype(jnp.int32), (1, 0))
    out5 = _gather_call()(tab_t, cond_t)
    out = jnp.transpose(out5, (2, 4, 0, 1, 3)).reshape(
        BATCH, N_DOMAIN, 8, 8)
    return out


# speedup vs baseline: 1.0329x; 1.0329x over previous
import functools

import jax
import jax.numpy as jnp
from jax import lax
from jax.experimental import pallas as pl
from jax.experimental.pallas import tpu as pltpu
from jax.experimental.pallas import tpu_sc as plsc

N_DOMAIN = 26
VOCAB = 100000
DIM = 64
BATCH = 16384
NC, NS, L = 2, 16, 16
NW = NC * NS
PAIRS = N_DOMAIN * DIM
P_PER_W = PAIRS // NW
HALF_B = BATCH // 2


QVEC = BATCH // 4 // L


def _sc_body(tab_hbm, cond_hbm, out_hbm, row_v, idx_v, outc0, outc1,
             sem_r, sem_i, sem_o0, sem_o1):
    wid = lax.axis_index("s") * NC + lax.axis_index("c")
    pair0 = wid * P_PER_W
    outc = (outc0, outc1)
    sems = (sem_o0, sem_o1)

    def issue_row(pair, first):
        i = pair // DIM
        d = pair - i * DIM

        @pl.when((d == 0) | first)
        def _():
            pltpu.async_copy(cond_hbm.at[i, :], idx_v, sem_i).wait()

        pltpu.async_copy(tab_hbm.at[i, d, :], row_v, sem_r)

    issue_row(pair0, True)

    def do_pair(p, _):
        pair = pair0 + p
        i = pair // DIM
        d = pair - i * DIM
        h = d // 8
        w = d - h * 8

        pltpu.make_async_copy(tab_hbm.at[i, d, :], row_v, sem_r).wait()

        handles = [None, None]
        for q in range(4):
            buf = outc[q % 2]
            if handles[q % 2] is not None:
                handles[q % 2].wait()

            @plsc.parallel_loop(0, 32, unroll=4)
            def brow(r, _q=q, _buf=buf):
                base = _q * (QVEC * L) + r * 128
                for k in range(8):
                    v = idx_v[pl.ds(base + k * L, L)]
                    g = plsc.load_gather(row_v, [v])
                    _buf[r, pl.ds(k * L, L)] = g
            handles[q % 2] = pltpu.async_copy(
                buf, out_hbm.at[i, h, pl.ds(q * 32, 32), w, :], sems[q % 2])

        @pl.when(p + 1 < P_PER_W)
        def _():
            issue_row(pair + 1, False)

        handles[0].wait()
        handles[1].wait()
        return 0

    lax.fori_loop(0, P_PER_W, do_pair, 0)


@functools.cache
def _gather_call():
    return functools.partial(
        pl.kernel,
        mesh=plsc.VectorSubcoreMesh(core_axis_name="c", subcore_axis_name="s",
                                    num_cores=NC),
        out_type=jax.ShapeDtypeStruct((N_DOMAIN, 8, 128, 8, 128), jnp.float32),
        compiler_params=pltpu.CompilerParams(use_tc_tiling_on_sc=True,
                                             needs_layout_passes=False),
        scratch_types=[
            pltpu.VMEM((VOCAB,), jnp.float32),
            pltpu.VMEM((BATCH,), jnp.int32),
            pltpu.VMEM((32, 128), jnp.float32),
            pltpu.VMEM((32, 128), jnp.float32),
            pltpu.SemaphoreType.DMA,
            pltpu.SemaphoreType.DMA,
            pltpu.SemaphoreType.DMA,
            pltpu.SemaphoreType.DMA,
        ],
    )(_sc_body)


def kernel(conditions, tables):
    tab_t = jnp.transpose(tables, (0, 2, 1))
    cond_t = jnp.transpose(conditions.astype(jnp.int32), (1, 0))
    out5 = _gather_call()(tab_t, cond_t)
    out = jnp.transpose(out5, (2, 4, 0, 1, 3)).reshape(
        BATCH, N_DOMAIN, 8, 8)
    return out


# speedup vs baseline: 1.0446x; 1.0113x over previous
import functools

import jax
import jax.numpy as jnp
from jax import lax
from jax.experimental import pallas as pl
from jax.experimental.pallas import tpu as pltpu
from jax.experimental.pallas import tpu_sc as plsc

N_DOMAIN = 26
VOCAB = 100000
DIM = 64
BATCH = 16384
NC, NS, L = 2, 16, 16
NW = NC * NS
PAIRS = N_DOMAIN * DIM
P_PER_W = PAIRS // NW
HALF_B = BATCH // 2


QVEC = BATCH // 4 // L


def _sc_body(tab_hbm, cond_hbm, out_hbm, row_v, idx_v, outc0, outc1,
             sem_r, sem_i, sem_o0, sem_o1):
    wid = lax.axis_index("s") * NC + lax.axis_index("c")
    pair0 = wid * P_PER_W
    outc = (outc0, outc1)
    sems = (sem_o0, sem_o1)

    def issue_row(pair, first):
        i = pair // DIM
        d = pair - i * DIM

        @pl.when((d == 0) | first)
        def _():
            pltpu.async_copy(cond_hbm.at[i, :], idx_v, sem_i).wait()

        pltpu.async_copy(tab_hbm.at[i, d, :], row_v, sem_r)

    issue_row(pair0, True)

    def do_pair(p, _):
        pair = pair0 + p
        i = pair // DIM
        d = pair - i * DIM
        h = d // 8
        w = d - h * 8

        pltpu.make_async_copy(tab_hbm.at[i, d, :], row_v, sem_r).wait()

        handles = [None, None]
        for q in range(4):
            buf = outc[q % 2]
            if handles[q % 2] is not None:
                handles[q % 2].wait()

            @plsc.parallel_loop(0, 32, unroll=8)
            def brow(r, _q=q, _buf=buf):
                base = _q * (QVEC * L) + r * 128
                for k in range(8):
                    v = idx_v[pl.ds(base + k * L, L)]
                    g = plsc.load_gather(row_v, [v])
                    _buf[r, pl.ds(k * L, L)] = g
            handles[q % 2] = pltpu.async_copy(
                buf, out_hbm.at[i, h, pl.ds(q * 32, 32), w, :], sems[q % 2])

        @pl.when(p + 1 < P_PER_W)
        def _():
            issue_row(pair + 1, False)

        handles[0].wait()
        handles[1].wait()
        return 0

    lax.fori_loop(0, P_PER_W, do_pair, 0)


@functools.cache
def _gather_call():
    return functools.partial(
        pl.kernel,
        mesh=plsc.VectorSubcoreMesh(core_axis_name="c", subcore_axis_name="s",
                                    num_cores=NC),
        out_type=jax.ShapeDtypeStruct((N_DOMAIN, 8, 128, 8, 128), jnp.float32),
        compiler_params=pltpu.CompilerParams(use_tc_tiling_on_sc=True,
                                             needs_layout_passes=False),
        scratch_types=[
            pltpu.VMEM((VOCAB,), jnp.float32),
            pltpu.VMEM((BATCH,), jnp.int32),
            pltpu.VMEM((32, 128), jnp.float32),
            pltpu.VMEM((32, 128), jnp.float32),
            pltpu.SemaphoreType.DMA,
            pltpu.SemaphoreType.DMA,
            pltpu.SemaphoreType.DMA,
            pltpu.SemaphoreType.DMA,
        ],
    )(_sc_body)


def kernel(conditions, tables):
    tab_t = jnp.transpose(tables, (0, 2, 1))
    cond_t = jnp.transpose(conditions.astype(jnp.int32), (1, 0))
    out5 = _gather_call()(tab_t, cond_t)
    out = jnp.transpose(out5, (2, 4, 0, 1, 3)).reshape(
        BATCH, N_DOMAIN, 8, 8)
    return out
